# baseline (speedup 1.0000x reference)
import functools

import jax
import jax.numpy as jnp
from jax import lax
from jax.experimental import pallas as pl
from jax.experimental.pallas import tpu as pltpu
from jax.experimental.pallas import tpu_sc as plsc

_ANCHOR_W = (116.0, 156.0, 373.0)
_ANCHOR_H = (90.0, 198.0, 326.0)
_CIN = 89
_COUT = 85
_G = 52
_RCHUNK = 8
_P = _RCHUNK * _G
_FULL_CHUNKS = _G // _RCHUNK
_TAIL_R0 = _FULL_CHUNKS * _RCHUNK
_TAIL_ROWS = _G - _TAIL_R0
_NA = 56
_NB = _CIN - _NA


def _sigmoid(v):
    return 1.0 / (1.0 + jnp.exp(-v))


def _sc_body(nslab, x_hbm, st_hbm, out_hbm, buf, obuf, stv, sem):
    nc = 2
    wid = lax.axis_index("s") * nc + lax.axis_index("c")
    pltpu.sync_copy(st_hbm, stv)
    i16 = lax.iota(jnp.int32, 16)

    def process_chunk(slab, r0, nrows):
        npos = nrows * _G
        nv = npos // 16
        b = slab // 3
        a = slab - b * 3
        aw = jnp.where(a == 0, _ANCHOR_W[0],
                       jnp.where(a == 1, _ANCHOR_W[1], _ANCHOR_W[2]))
        ah = jnp.where(a == 0, _ANCHOR_H[0],
                       jnp.where(a == 1, _ANCHOR_H[1], _ANCHOR_H[2]))
        st = stv[:]

        dst_a = buf.at[pl.ds(0, _NA)] if nrows == _RCHUNK \
            else buf.at[pl.ds(0, _NA), pl.ds(0, nrows)]
        pltpu.sync_copy(
            x_hbm.at[b, pl.ds(a * _CIN, _NA), pl.ds(r0, nrows), :], dst_a)

        def box_body(j, carry2):
            p = i16 + j * 16
            rv = p // _G
            lv = p - rv * _G
            s0 = _sigmoid(plsc.load_gather(buf, [jnp.full(16, 0, jnp.int32), rv, lv]))
            s1 = _sigmoid(plsc.load_gather(buf, [jnp.full(16, 1, jnp.int32), rv, lv]))
            e2 = jnp.exp(plsc.load_gather(buf, [jnp.full(16, 2, jnp.int32), rv, lv]))
            e3 = jnp.exp(plsc.load_gather(buf, [jnp.full(16, 3, jnp.int32), rv, lv]))
            plsc.store_scatter(obuf, [p, jnp.full(16, 0, jnp.int32)],
                               (s0 + lv.astype(jnp.float32)) * st)
            plsc.store_scatter(obuf, [p, jnp.full(16, 1, jnp.int32)],
                               (s1 + (rv + r0).astype(jnp.float32)) * st)
            plsc.store_scatter(obuf, [p, jnp.full(16, 2, jnp.int32)], e2 * aw)
            plsc.store_scatter(obuf, [p, jnp.full(16, 3, jnp.int32)], e3 * ah)
            return carry2

        lax.fori_loop(0, nv, box_body, 0)

        def clsa_body(p, carry2):
            r = p // _G
            rv = jnp.full(16, r, jnp.int32)
            lv = jnp.full(16, p - r * _G, jnp.int32)
            pv = jnp.full(16, p, jnp.int32)
            for src0, c0 in ((8, 4), (24, 20), (40, 36)):
                v = _sigmoid(plsc.load_gather(buf, [i16 + src0, rv, lv]))
                plsc.store_scatter(obuf, [pv, i16 + c0], v)
            return carry2

        lax.fori_loop(0, npos, clsa_body, 0)

        dst_b = buf.at[pl.ds(0, _NB)] if nrows == _RCHUNK \
            else buf.at[pl.ds(0, _NB), pl.ds(0, nrows)]
        pltpu.sync_copy(
            x_hbm.at[b, pl.ds(a * _CIN + _NA, _NB), pl.ds(r0, nrows), :], dst_b)

        def clsb_body(p, carry2):
            r = p // _G
            rv = jnp.full(16, r, jnp.int32)
            lv = jnp.full(16, p - r * _G, jnp.int32)
            pv = jnp.full(16, p, jnp.int32)
            for src0, c0 in ((0, 52), (16, 68)):
                v = _sigmoid(plsc.load_gather(buf, [i16 + src0, rv, lv]))
                plsc.store_scatter(obuf, [pv, i16 + c0], v)
            return carry2

        lax.fori_loop(0, npos, clsb_body, 0)

        def c84_body(j, carry2):
            p = i16 + j * 16
            rv = p // _G
            lv = p - rv * _G
            v = _sigmoid(plsc.load_gather(buf, [jnp.full(16, 32, jnp.int32), rv, lv]))
            plsc.store_scatter(obuf, [p, jnp.full(16, 84, jnp.int32)], v)
            return carry2

        lax.fori_loop(0, nv, c84_body, 0)

        n0 = a * (_G * _G) + r0 * _G
        src_o = obuf if nrows == _RCHUNK else obuf.at[pl.ds(0, npos)]
        pltpu.sync_copy(src_o, out_hbm.at[b, pl.ds(n0, npos), :])

    nfull = nslab * _FULL_CHUNKS

    def full_body(t, carry):
        chunk = wid + t * 32

        @pl.when(chunk < nfull)
        def _():
            slab = chunk // _FULL_CHUNKS
            k = chunk - slab * _FULL_CHUNKS
            process_chunk(slab, k * _RCHUNK, _RCHUNK)

        return carry

    lax.fori_loop(0, (nfull + 31) // 32, full_body, 0)

    def tail_body(t, carry):
        slab = wid + t * 32

        @pl.when(slab < nslab)
        def _():
            process_chunk(slab, _TAIL_R0, _TAIL_ROWS)

        return carry

    lax.fori_loop(0, (nslab + 31) // 32, tail_body, 0)


def kernel(x, img_dim):
    B = x.shape[0]
    g = x.shape[2]
    s = g * g
    st = jnp.asarray(img_dim, jnp.float32) / g
    st16 = jnp.broadcast_to(st, (16,))

    nslab = B * 3
    mesh = plsc.VectorSubcoreMesh(core_axis_name="c", subcore_axis_name="s")
    sc = functools.partial(
        pl.kernel,
        mesh=mesh,
        compiler_params=pltpu.CompilerParams(needs_layout_passes=False),
        out_type=jax.ShapeDtypeStruct((B, 3 * s, _COUT), jnp.float32),
        scratch_types=[
            pltpu.VMEM((_NA, _RCHUNK, g), jnp.float32),
            pltpu.VMEM((_P, _COUT), jnp.float32),
            pltpu.VMEM((16,), jnp.float32),
            pltpu.SemaphoreType.DMA,
        ],
    )(functools.partial(_sc_body, nslab))
    out = sc(x, st16)
    return (out, 0)


# speedup vs baseline: 2.0847x; 2.0847x over previous
import functools

import jax
import jax.numpy as jnp
from jax import lax
from jax.experimental import pallas as pl
from jax.experimental.pallas import tpu as pltpu
from jax.experimental.pallas import tpu_sc as plsc

_ANCHOR_W = (116.0, 156.0, 373.0)
_ANCHOR_H = (90.0, 198.0, 326.0)
_CIN = 89
_COUT = 85
_G = 52
_RCHUNK = 8
_HP = 4 * _G
_FULL_CHUNKS = _G // _RCHUNK
_TAIL_R0 = _FULL_CHUNKS * _RCHUNK
_NA = 56
_NB = _CIN - _NA


def _sigmoid(v):
    return 1.0 / (1.0 + jnp.exp(-v))


def _sc_body(nslab, x_hbm, st_hbm, out_hbm, bufa, bufb, obuf, stv, sem):
    nc = 2
    wid = lax.axis_index("s") * nc + lax.axis_index("c")
    pltpu.sync_copy(st_hbm, stv)
    i16 = lax.iota(jnp.int32, 16)

    def process_chunk(slab, r0, nrows):
        b = slab // 3
        a = slab - b * 3
        aw = jnp.where(a == 0, _ANCHOR_W[0],
                       jnp.where(a == 1, _ANCHOR_W[1], _ANCHOR_W[2]))
        ah = jnp.where(a == 0, _ANCHOR_H[0],
                       jnp.where(a == 1, _ANCHOR_H[1], _ANCHOR_H[2]))
        st = stv[:]

        dst_a = bufa if nrows == _RCHUNK else bufa.at[:, pl.ds(0, nrows)]
        dst_b = bufb if nrows == _RCHUNK else bufb.at[:, pl.ds(0, nrows)]
        ha = pltpu.async_copy(
            x_hbm.at[b, pl.ds(a * _CIN, _NA), pl.ds(r0, nrows), :], dst_a, sem)
        hb = pltpu.async_copy(
            x_hbm.at[b, pl.ds(a * _CIN + _NA, _NB), pl.ds(r0, nrows), :],
            dst_b, sem)
        ha.wait()
        hb.wait()

        for h in range(nrows // 4):
            p0 = h * _HP

            @plsc.parallel_loop(0, _HP, 1, unroll=2)
            def cls_loop(p):
                pg = p + p0
                r = pg // _G
                rv = jnp.full(16, r, jnp.int32)
                lv = jnp.full(16, pg - r * _G, jnp.int32)
                pv = jnp.full(16, p, jnp.int32)
                for src0, c0 in ((8, 4), (24, 20), (40, 36)):
                    v = _sigmoid(plsc.load_gather(bufa, [i16 + src0, rv, lv]))
                    plsc.store_scatter(obuf, [pv, i16 + c0], v)
                for src0, c0 in ((0, 52), (16, 68)):
                    v = _sigmoid(plsc.load_gather(bufb, [i16 + src0, rv, lv]))
                    plsc.store_scatter(obuf, [pv, i16 + c0], v)

            @plsc.parallel_loop(0, _HP // 16, 1, unroll=2)
            def box_loop(j):
                p = i16 + j * 16
                pg = p + p0
                rv = pg // _G
                lv = pg - rv * _G
                z = jnp.full(16, 0, jnp.int32)
                s0 = _sigmoid(plsc.load_gather(bufa, [z, rv, lv]))
                s1 = _sigmoid(plsc.load_gather(bufa, [z + 1, rv, lv]))
                e2 = jnp.exp(plsc.load_gather(bufa, [z + 2, rv, lv]))
                e3 = jnp.exp(plsc.load_gather(bufa, [z + 3, rv, lv]))
                c84 = _sigmoid(plsc.load_gather(bufb, [z + 32, rv, lv]))
                plsc.store_scatter(obuf, [p, z],
                                   (s0 + lv.astype(jnp.float32)) * st)
                plsc.store_scatter(obuf, [p, z + 1],
                                   (s1 + (rv + r0).astype(jnp.float32)) * st)
                plsc.store_scatter(obuf, [p, z + 2], e2 * aw)
                plsc.store_scatter(obuf, [p, z + 3], e3 * ah)
                plsc.store_scatter(obuf, [p, z + 84], c84)

            n0 = a * (_G * _G) + r0 * _G + p0
            pltpu.sync_copy(obuf, out_hbm.at[b, pl.ds(n0, _HP), :])

    nfull = nslab * _FULL_CHUNKS

    def full_body(t, carry):
        chunk = wid + t * 32
        slab = chunk // _FULL_CHUNKS
        k = chunk - slab * _FULL_CHUNKS
        process_chunk(slab, k * _RCHUNK, _RCHUNK)
        return carry

    lax.fori_loop(0, nfull // 32, full_body, 0)

    def tail_body(t, carry):
        slab = wid + t * 32

        @pl.when(slab < nslab)
        def _():
            process_chunk(slab, _TAIL_R0, _G - _TAIL_R0)

        return carry

    lax.fori_loop(0, (nslab + 31) // 32, tail_body, 0)


def kernel(x, img_dim):
    B = x.shape[0]
    g = x.shape[2]
    s = g * g
    st = jnp.asarray(img_dim, jnp.float32) / g
    st16 = jnp.broadcast_to(st, (16,))

    nslab = B * 3
    mesh = plsc.VectorSubcoreMesh(core_axis_name="c", subcore_axis_name="s")
    sc = functools.partial(
        pl.kernel,
        mesh=mesh,
        compiler_params=pltpu.CompilerParams(needs_layout_passes=False),
        out_type=jax.ShapeDtypeStruct((B, 3 * s, _COUT), jnp.float32),
        scratch_types=[
            pltpu.VMEM((_NA, _RCHUNK, g), jnp.float32),
            pltpu.VMEM((_NB, _RCHUNK, g), jnp.float32),
            pltpu.VMEM((_HP, _COUT), jnp.float32),
            pltpu.VMEM((16,), jnp.float32),
            pltpu.SemaphoreType.DMA,
        ],
    )(functools.partial(_sc_body, nslab))
    out = sc(x, st16)
    return (out, 0)
